# baseline (speedup 1.0000x reference)
import jax
import jax.numpy as jnp
from jax import lax
from jax.experimental import pallas as pl
from jax.experimental.pallas import tpu as pltpu

_T = 64
_U = 16
_D = 1024
_NEG = -1e30


def _shd(x, k, fill):
    pad = jnp.full((k,) + x.shape[1:], fill, x.dtype)
    return jnp.concatenate([pad, x[: x.shape[0] - k]], axis=0)


def _body(x_ref, tgt_ref, tl_ref, o_lp, o_la, o_st, o_tot):
    x = x_ref[0]
    tl = tl_ref[0]

    m = jnp.max(x, axis=2)
    e = jnp.exp(x - m[:, :, None])
    logs = jnp.log(jnp.sum(e, axis=2))
    colio = lax.broadcasted_iota(jnp.int32, (_U, _D), 1)
    mask = colio == tgt_ref[:].reshape(1, _U).T
    g_raw = jnp.sum(jnp.where(mask[None], x, 0.0), axis=2)
    gath = (g_raw - m) - logs
    bl = (x[:, :, 0] - m) - logs

    z = bl
    for k in (1, 2, 4, 8, 16, 32):
        z = z + _shd(z, k, 0.0)
    bc = _shd(z, 1, 0.0)

    tar = lax.broadcasted_iota(jnp.int32, (_T, 1), 0).astype(jnp.float32)

    gcol = [gath[:, u : u + 1] for u in range(_U)]
    bccol = [bc[:, u : u + 1] for u in range(_U)]
    blcol = [bl[:, u : u + 1] for u in range(_U)]

    la = jnp.zeros((_T, 1), jnp.float32)
    st = tar
    tot = jnp.ones((_T, 1), jnp.float32)
    acc_la, acc_st, acc_tot = la, st, tot
    acc_bl = blcol[0]

    for u in range(1, _U):
        v = la + (gcol[u - 1] - bccol[u])
        p_st = st
        p_tot = tot - tar
        for k in (1, 2, 4, 8, 16, 32):
            vs = _shd(v, k, _NEG)
            ss = _shd(p_st, k, 0.0)
            ts = _shd(p_tot, k, 0.0)
            keep = vs >= v
            v = jnp.where(keep, vs, v)
            p_st = jnp.where(keep, ss, p_st)
            p_tot = jnp.where(keep, ts, p_tot)
        la = v + bccol[u]
        st = p_st
        tot = p_tot + tar + 1.0
        pred = tl == u
        acc_la = jnp.where(pred, la, acc_la)
        acc_st = jnp.where(pred, st, acc_st)
        acc_tot = jnp.where(pred, tot, acc_tot)
        acc_bl = jnp.where(pred, blcol[u], acc_bl)

    la_each = acc_la + acc_bl
    o_lp[:] = la_each[_T - 1 :, :]
    o_la[:] = la_each
    o_st[:] = acc_st
    o_tot[:] = acc_tot + 1.0


def kernel(logits, targets, logit_lens, target_lens):
    tl = target_lens.astype(jnp.int32)

    out_shape = [
        jax.ShapeDtypeStruct((1, 1), jnp.float32),
        jax.ShapeDtypeStruct((_T, 1), jnp.float32),
        jax.ShapeDtypeStruct((_T, 1), jnp.float32),
        jax.ShapeDtypeStruct((_T, 1), jnp.float32),
    ]
    lp, la_each, st_each, tot_each = pl.pallas_call(
        _body,
        out_shape=out_shape,
        in_specs=[
            pl.BlockSpec((1, _T, _U, _D), lambda: (0, 0, 0, 0)),
            pl.BlockSpec((1, _U), lambda: (0, 0)),
            pl.BlockSpec(memory_space=pltpu.SMEM),
        ],
        out_specs=[
            pl.BlockSpec((1, 1), lambda: (0, 0)),
            pl.BlockSpec((_T, 1), lambda: (0, 0)),
            pl.BlockSpec((_T, 1), lambda: (0, 0)),
            pl.BlockSpec((_T, 1), lambda: (0, 0)),
        ],
    )(logits, targets.astype(jnp.int32), tl)

    return (
        lp.reshape(()),
        la_each.reshape(_T),
        st_each.reshape(_T),
        tot_each.reshape(_T),
    )
